# Initial kernel scaffold; baseline (speedup 1.0000x reference)
import jax
import jax.numpy as jnp
from jax.experimental import pallas as pl


def kernel(pos, classes, edges, batch_nodes, W_in, b_in, W1, b1, W2, b2, W3, b3, Wih_n, bih_n, Whh_n, bhh_n, Wih_e, bih_e, Whh_e, bhh_e):
    raise NotImplementedError("write your pallas kernel here")


# speedup vs baseline: 3.5657x; 3.5657x over previous
import functools

import jax
import jax.numpy as jnp
from jax import lax
from jax.experimental import pallas as pl
from jax.experimental.pallas import tpu as pltpu
from jax.experimental.pallas import tpu_sc as plsc

N_ITERS = 6
BE = 2000
BN = 2000
_CHUNK = 128
_GRP = 2


def _sc_info():
    info = plsc.get_sparse_core_info()
    return info.num_cores, info.num_subcores


@functools.lru_cache(maxsize=None)
def _make_gather(d, n_rows, n_chunks):
    nc, ns = _sc_info()
    n_groups = n_chunks // _GRP
    gpw = (n_groups // nc + ns - 1) // ns
    mesh = plsc.VectorSubcoreMesh(core_axis_name="c", subcore_axis_name="s")

    @functools.partial(
        pl.kernel,
        mesh=mesh,
        out_type=jax.ShapeDtypeStruct((n_rows, d), jnp.float32),
        scratch_types=[
            pltpu.VMEM((_GRP, _CHUNK), jnp.int32),
            pltpu.VMEM((_GRP * _CHUNK, d), jnp.float32),
            pltpu.SemaphoreType.DMA,
        ],
        compiler_params=pltpu.CompilerParams(use_tc_tiling_on_sc=False),
    )
    def gather_k(table_hbm, idx_hbm, out_hbm, idx_v, rows_v, sem):
        c = lax.axis_index("c")
        sid = lax.axis_index("s")

        def body(j, carry):
            g = (sid + j * ns) * nc + c

            @pl.when(g < n_groups)
            def _():
                pltpu.sync_copy(idx_hbm.at[pl.ds(g * _GRP, _GRP)], idx_v)
                for b in range(_GRP):
                    pltpu.sync_copy(table_hbm.at[idx_v.at[b]],
                                    rows_v.at[pl.ds(b * _CHUNK, _CHUNK)])
                pltpu.sync_copy(
                    rows_v, out_hbm.at[pl.ds(g * _GRP * _CHUNK, _GRP * _CHUNK)])

            return carry

        lax.fori_loop(0, gpw, body, 0)

    return gather_k


def _in_linear_body(c_ref, w_ref, b_ref, o_ref):
    o_ref[...] = (
        jnp.dot(c_ref[...], w_ref[...], preferred_element_type=jnp.float32)
        + b_ref[...]
    )


def _in_linear(classes, W_in, b_in):
    n, c = classes.shape
    d = W_in.shape[1]
    return pl.pallas_call(
        _in_linear_body,
        grid=(n // BN,),
        in_specs=[
            pl.BlockSpec((BN, c), lambda i: (i, 0)),
            pl.BlockSpec((c, d), lambda i: (0, 0)),
            pl.BlockSpec((1, d), lambda i: (0, 0)),
        ],
        out_specs=pl.BlockSpec((BN, d), lambda i: (i, 0)),
        out_shape=jax.ShapeDtypeStruct((n, d), jnp.float32),
    )(classes, W_in, b_in.reshape(1, d))


def _dpos_body(pu_ref, pv_ref, o_ref):
    o_ref[...] = (pv_ref[...] - pu_ref[...])[:, :3]


def _dpos(p, e):
    nb = e // BE
    return pl.pallas_call(
        _dpos_body,
        grid=(nb,),
        in_specs=[
            pl.BlockSpec((BE, 16), lambda b: (b, 0)),
            pl.BlockSpec((BE, 16), lambda b: (nb + b, 0)),
        ],
        out_specs=pl.BlockSpec((BE, 3), lambda b: (b, 0)),
        out_shape=jax.ShapeDtypeStruct((e, 3), jnp.float32),
    )(p, p)


def _mlp_body(fu_ref, fv_ref, dp_ref, w1_ref, b1_ref,
              w2_ref, b2_ref, w3_ref, b3_ref, of_ref, ob_ref):
    fu = fu_ref[...]
    fv = fv_ref[...]
    dp = dp_ref[...]
    for o_ref, inputs in (
        (of_ref, jnp.concatenate([fu, fv, dp], axis=1)),
        (ob_ref, jnp.concatenate([fv, fu, -dp], axis=1)),
    ):
        h1 = jax.nn.relu(
            jnp.dot(inputs, w1_ref[...], preferred_element_type=jnp.float32)
            + b1_ref[...]
        )
        h2 = jax.nn.relu(
            jnp.dot(h1, w2_ref[...], preferred_element_type=jnp.float32)
            + b2_ref[...]
        )
        o_ref[...] = (
            jnp.dot(h2, w3_ref[...], preferred_element_type=jnp.float32)
            + b3_ref[...]
        )


def _mlp(g, dpos, W1, b1, W2, b2, W3, b3, e):
    d = g.shape[1]
    k = W1.shape[0]
    h = W2.shape[0]
    m = W3.shape[1]
    nb = e // BE
    return pl.pallas_call(
        _mlp_body,
        grid=(nb,),
        in_specs=[
            pl.BlockSpec((BE, d), lambda b: (b, 0)),
            pl.BlockSpec((BE, d), lambda b: (nb + b, 0)),
            pl.BlockSpec((BE, 3), lambda b: (b, 0)),
            pl.BlockSpec((k, h), lambda b: (0, 0)),
            pl.BlockSpec((1, h), lambda b: (0, 0)),
            pl.BlockSpec((h, h), lambda b: (0, 0)),
            pl.BlockSpec((1, h), lambda b: (0, 0)),
            pl.BlockSpec((h, m), lambda b: (0, 0)),
            pl.BlockSpec((1, m), lambda b: (0, 0)),
        ],
        out_specs=[
            pl.BlockSpec((BE, m), lambda b: (b, 0)),
            pl.BlockSpec((BE, m), lambda b: (b, 0)),
        ],
        out_shape=[
            jax.ShapeDtypeStruct((e, m), jnp.float32),
            jax.ShapeDtypeStruct((e, m), jnp.float32),
        ],
    )(g, g, dpos, W1, b1.reshape(1, h),
      W2, b2.reshape(1, h), W3, b3.reshape(1, m))


def _gru_body(a_ref, h_ref, wih_ref, bih_ref, whh_ref, bhh_ref, o_ref):
    d = h_ref.shape[1]
    gi = jnp.dot(a_ref[...], wih_ref[...], preferred_element_type=jnp.float32) + bih_ref[...]
    gh = jnp.dot(h_ref[...], whh_ref[...], preferred_element_type=jnp.float32) + bhh_ref[...]
    r = jax.nn.sigmoid(gi[:, :d] + gh[:, :d])
    z = jax.nn.sigmoid(gi[:, d:2 * d] + gh[:, d:2 * d])
    nn = jnp.tanh(gi[:, 2 * d:] + r * gh[:, 2 * d:])
    o_ref[...] = (1.0 - z) * nn + z * h_ref[...]


def _gru(a, h, Wih, bih, Whh, bhh):
    n, d = h.shape
    g = Wih.shape[1]
    return pl.pallas_call(
        _gru_body,
        grid=(n // BN,),
        in_specs=[
            pl.BlockSpec((BN, d), lambda i: (i, 0)),
            pl.BlockSpec((BN, d), lambda i: (i, 0)),
            pl.BlockSpec((d, g), lambda i: (0, 0)),
            pl.BlockSpec((1, g), lambda i: (0, 0)),
            pl.BlockSpec((d, g), lambda i: (0, 0)),
            pl.BlockSpec((1, g), lambda i: (0, 0)),
        ],
        out_specs=pl.BlockSpec((BN, d), lambda i: (i, 0)),
        out_shape=jax.ShapeDtypeStruct((n, d), jnp.float32),
    )(a, h, Wih, bih.reshape(1, g), Whh, bhh.reshape(1, g))


def kernel(pos, classes, edges, batch_nodes, W_in, b_in, W1, b1, W2, b2,
           W3, b3, Wih_n, bih_n, Whh_n, bhh_n, Wih_e, bih_e, Whh_e, bhh_e):
    n = classes.shape[0]
    e = edges.shape[1]
    src = edges[0]
    dst = edges[1]

    gidx = jnp.concatenate([src, dst]).reshape(-1, _CHUNK)
    n_chunks = gidx.shape[0]

    pos16 = jnp.pad(pos, ((0, 0), (0, 13)))
    p = _make_gather(16, 2 * e, n_chunks)(pos16, gidx)
    dpos = _dpos(p, e)

    nf = _in_linear(classes, W_in, b_in)
    gather32 = _make_gather(32, 2 * e, n_chunks)

    for _ in range(N_ITERS):
        g = gather32(nf, gidx)
        mf, mb = _mlp(g, dpos, W1, b1, W2, b2, W3, b3, e)
        a = jnp.zeros((n, 32), jnp.float32).at[src].add(mf).at[dst].add(mb)
        nf = _gru(a, nf, Wih_n, bih_n, Whh_n, bhh_n)

    return nf[None]


# speedup vs baseline: 3.5892x; 1.0066x over previous
import functools

import jax
import jax.numpy as jnp
from jax import lax
from jax.experimental import pallas as pl
from jax.experimental.pallas import tpu as pltpu
from jax.experimental.pallas import tpu_sc as plsc

N_ITERS = 6
BE = 2000
BN = 2000
_CHUNK = 128
_GRP = 4


def _sc_info():
    info = plsc.get_sparse_core_info()
    return info.num_cores, info.num_subcores


@functools.lru_cache(maxsize=None)
def _make_gather(d, n_rows, n_chunks):
    nc, ns = _sc_info()
    n_groups = n_chunks // _GRP
    gpw = (n_groups // nc + ns - 1) // ns
    mesh = plsc.VectorSubcoreMesh(core_axis_name="c", subcore_axis_name="s")

    @functools.partial(
        pl.kernel,
        mesh=mesh,
        out_type=jax.ShapeDtypeStruct((n_rows, d), jnp.float32),
        scratch_types=[
            pltpu.VMEM((_GRP, _CHUNK), jnp.int32),
            pltpu.VMEM((_GRP * _CHUNK, d), jnp.float32),
            pltpu.SemaphoreType.DMA,
        ],
        compiler_params=pltpu.CompilerParams(use_tc_tiling_on_sc=False),
    )
    def gather_k(table_hbm, idx_hbm, out_hbm, idx_v, rows_v, sem):
        c = lax.axis_index("c")
        sid = lax.axis_index("s")

        def body(j, carry):
            g = (sid + j * ns) * nc + c

            @pl.when(g < n_groups)
            def _():
                pltpu.sync_copy(idx_hbm.at[pl.ds(g * _GRP, _GRP)], idx_v)
                for b in range(_GRP):
                    pltpu.sync_copy(table_hbm.at[idx_v.at[b]],
                                    rows_v.at[pl.ds(b * _CHUNK, _CHUNK)])
                pltpu.sync_copy(
                    rows_v, out_hbm.at[pl.ds(g * _GRP * _CHUNK, _GRP * _CHUNK)])

            return carry

        lax.fori_loop(0, gpw, body, 0)

    return gather_k


def _in_linear_body(c_ref, w_ref, b_ref, o_ref):
    o_ref[...] = (
        jnp.dot(c_ref[...], w_ref[...], preferred_element_type=jnp.float32)
        + b_ref[...]
    )


def _in_linear(classes, W_in, b_in):
    n, c = classes.shape
    d = W_in.shape[1]
    return pl.pallas_call(
        _in_linear_body,
        grid=(n // BN,),
        in_specs=[
            pl.BlockSpec((BN, c), lambda i: (i, 0)),
            pl.BlockSpec((c, d), lambda i: (0, 0)),
            pl.BlockSpec((1, d), lambda i: (0, 0)),
        ],
        out_specs=pl.BlockSpec((BN, d), lambda i: (i, 0)),
        out_shape=jax.ShapeDtypeStruct((n, d), jnp.float32),
    )(classes, W_in, b_in.reshape(1, d))


def _dpos_body(pu_ref, pv_ref, o_ref):
    o_ref[...] = (pv_ref[...] - pu_ref[...])[:, :3]


def _dpos(p, e):
    nb = e // BE
    return pl.pallas_call(
        _dpos_body,
        grid=(nb,),
        in_specs=[
            pl.BlockSpec((BE, 16), lambda b: (b, 0)),
            pl.BlockSpec((BE, 16), lambda b: (nb + b, 0)),
        ],
        out_specs=pl.BlockSpec((BE, 3), lambda b: (b, 0)),
        out_shape=jax.ShapeDtypeStruct((e, 3), jnp.float32),
    )(p, p)


def _mlp_body(fu_ref, fv_ref, dp_ref, w1_ref, b1_ref,
              w2_ref, b2_ref, w3_ref, b3_ref, of_ref, ob_ref):
    fu = fu_ref[...]
    fv = fv_ref[...]
    dp = dp_ref[...]
    for o_ref, inputs in (
        (of_ref, jnp.concatenate([fu, fv, dp], axis=1)),
        (ob_ref, jnp.concatenate([fv, fu, -dp], axis=1)),
    ):
        h1 = jax.nn.relu(
            jnp.dot(inputs, w1_ref[...], preferred_element_type=jnp.float32)
            + b1_ref[...]
        )
        h2 = jax.nn.relu(
            jnp.dot(h1, w2_ref[...], preferred_element_type=jnp.float32)
            + b2_ref[...]
        )
        o_ref[...] = (
            jnp.dot(h2, w3_ref[...], preferred_element_type=jnp.float32)
            + b3_ref[...]
        )


def _mlp(g, dpos, W1, b1, W2, b2, W3, b3, e):
    d = g.shape[1]
    k = W1.shape[0]
    h = W2.shape[0]
    m = W3.shape[1]
    nb = e // BE
    return pl.pallas_call(
        _mlp_body,
        grid=(nb,),
        in_specs=[
            pl.BlockSpec((BE, d), lambda b: (b, 0)),
            pl.BlockSpec((BE, d), lambda b: (nb + b, 0)),
            pl.BlockSpec((BE, 3), lambda b: (b, 0)),
            pl.BlockSpec((k, h), lambda b: (0, 0)),
            pl.BlockSpec((1, h), lambda b: (0, 0)),
            pl.BlockSpec((h, h), lambda b: (0, 0)),
            pl.BlockSpec((1, h), lambda b: (0, 0)),
            pl.BlockSpec((h, m), lambda b: (0, 0)),
            pl.BlockSpec((1, m), lambda b: (0, 0)),
        ],
        out_specs=[
            pl.BlockSpec((BE, m), lambda b: (b, 0)),
            pl.BlockSpec((BE, m), lambda b: (b, 0)),
        ],
        out_shape=[
            jax.ShapeDtypeStruct((e, m), jnp.float32),
            jax.ShapeDtypeStruct((e, m), jnp.float32),
        ],
    )(g, g, dpos, W1, b1.reshape(1, h),
      W2, b2.reshape(1, h), W3, b3.reshape(1, m))


def _gru_body(a_ref, h_ref, wih_ref, bih_ref, whh_ref, bhh_ref, o_ref):
    d = h_ref.shape[1]
    gi = jnp.dot(a_ref[...], wih_ref[...], preferred_element_type=jnp.float32) + bih_ref[...]
    gh = jnp.dot(h_ref[...], whh_ref[...], preferred_element_type=jnp.float32) + bhh_ref[...]
    r = jax.nn.sigmoid(gi[:, :d] + gh[:, :d])
    z = jax.nn.sigmoid(gi[:, d:2 * d] + gh[:, d:2 * d])
    nn = jnp.tanh(gi[:, 2 * d:] + r * gh[:, 2 * d:])
    o_ref[...] = (1.0 - z) * nn + z * h_ref[...]


def _gru(a, h, Wih, bih, Whh, bhh):
    n, d = h.shape
    g = Wih.shape[1]
    return pl.pallas_call(
        _gru_body,
        grid=(n // BN,),
        in_specs=[
            pl.BlockSpec((BN, d), lambda i: (i, 0)),
            pl.BlockSpec((BN, d), lambda i: (i, 0)),
            pl.BlockSpec((d, g), lambda i: (0, 0)),
            pl.BlockSpec((1, g), lambda i: (0, 0)),
            pl.BlockSpec((d, g), lambda i: (0, 0)),
            pl.BlockSpec((1, g), lambda i: (0, 0)),
        ],
        out_specs=pl.BlockSpec((BN, d), lambda i: (i, 0)),
        out_shape=jax.ShapeDtypeStruct((n, d), jnp.float32),
    )(a, h, Wih, bih.reshape(1, g), Whh, bhh.reshape(1, g))


def kernel(pos, classes, edges, batch_nodes, W_in, b_in, W1, b1, W2, b2,
           W3, b3, Wih_n, bih_n, Whh_n, bhh_n, Wih_e, bih_e, Whh_e, bhh_e):
    n = classes.shape[0]
    e = edges.shape[1]
    src = edges[0]
    dst = edges[1]

    gidx = jnp.concatenate([src, dst]).reshape(-1, _CHUNK)
    n_chunks = gidx.shape[0]

    pos16 = jnp.pad(pos, ((0, 0), (0, 13)))
    p = _make_gather(16, 2 * e, n_chunks)(pos16, gidx)
    dpos = _dpos(p, e)

    nf = _in_linear(classes, W_in, b_in)
    gather32 = _make_gather(32, 2 * e, n_chunks)

    for _ in range(N_ITERS):
        g = gather32(nf, gidx)
        mf, mb = _mlp(g, dpos, W1, b1, W2, b2, W3, b3, e)
        a = jnp.zeros((n, 32), jnp.float32).at[src].add(mf).at[dst].add(mb)
        nf = _gru(a, nf, Wih_n, bih_n, Whh_n, bhh_n)

    return nf[None]
